# baseline (speedup 1.0000x reference)
import jax
import jax.numpy as jnp
from jax import lax
from jax.experimental import pallas as pl
from jax.experimental.pallas import tpu as pltpu
from jax.experimental.pallas import tpu_sc as plsc

_B = 16384
_N = 1000000
_F = 32
_L = 16
_NC = 2
_NS = 16
_NW = _NC * _NS
_BPW = _B // _NW
_ICH = 128
_NCHUNK = _BPW // _ICH
_BROW = _L

_BLK = 32768
_NB1 = 16
_SC0 = _NB1 * _BLK
_CPW = 116
_SCLEN = _NW * _CPW * 128
_SCEND = _SC0 + _SCLEN
_TAIL = _N - _SCEND
_NAROWS = _N // _BROW
_NBROWS = _SCLEN // _BROW
_NCROWS = _TAIL // _BROW
_CBLK = 1024


def _dot_body(wt_ref, ht_ref, wb_ref, hb_ref, out_ref):
    out_ref[...] = (jnp.sum(wt_ref[...] * ht_ref[...], axis=0)
                    + wb_ref[...] + hb_ref[...])


_dot_scan_tc = pl.pallas_call(
    _dot_body,
    out_shape=jax.ShapeDtypeStruct((_SC0,), jnp.float32),
    grid=(_NB1,),
    in_specs=[
        pl.BlockSpec((_F, _BLK), lambda i: (0, i)),
        pl.BlockSpec((_F, _BLK), lambda i: (0, i)),
        pl.BlockSpec((_BLK,), lambda i: (i,)),
        pl.BlockSpec((_BLK,), lambda i: (i,)),
    ],
    out_specs=pl.BlockSpec((_BLK,), lambda i: (i,)),
)

_CIDX = _SCEND // _CBLK

_dot_scan_tail = pl.pallas_call(
    _dot_body,
    out_shape=jax.ShapeDtypeStruct((_TAIL,), jnp.float32),
    grid=(1,),
    in_specs=[
        pl.BlockSpec((_F, _CBLK), lambda i: (0, _CIDX)),
        pl.BlockSpec((_F, _CBLK), lambda i: (0, _CIDX)),
        pl.BlockSpec((_CBLK,), lambda i: (_CIDX,)),
        pl.BlockSpec((_CBLK,), lambda i: (_CIDX,)),
    ],
    out_specs=pl.BlockSpec((_CBLK,), lambda i: (0,)),
)

_mesh = plsc.VectorSubcoreMesh(core_axis_name="c", subcore_axis_name="s")


_CH = 256
_NSTEP = _CPW * 128 // _CH


def _scan_body(wt_hbm, ht_hbm, wb_hbm, hb_hbm, out_hbm,
               wb0, hb0, wv0, hv0, rs0,
               wb1, hb1, wv1, hv1, rs1, isem0, isem1, osem0, osem1):
    wid = lax.axis_index("s") * _NC + lax.axis_index("c")
    local0 = wid * _CPW * 128

    def _in_descs(step, wb, hb, wv, hv, sem):
        base = pl.multiple_of(_SC0 + local0 + step * _CH, 128)
        return (
            pltpu.make_async_copy(wt_hbm.at[:, pl.ds(base, _CH)], wb, sem),
            pltpu.make_async_copy(ht_hbm.at[:, pl.ds(base, _CH)], hb, sem),
            pltpu.make_async_copy(wb_hbm.at[pl.ds(base, _CH)], wv, sem),
            pltpu.make_async_copy(hb_hbm.at[pl.ds(base, _CH)], hv, sem),
        )

    def _out_desc(step, rs, sem):
        return pltpu.make_async_copy(
            rs, out_hbm.at[pl.ds(local0 + step * _CH, _CH)], sem)

    def _fire(descs):
        for d in descs:
            d.start()

    def _wait(descs):
        for d in descs:
            d.wait()

    def _compute(wb, hb, wv, hv, rs):
        for cg in range(_CH // _L):
            sl = pl.ds(cg * _L, _L)
            acc = wv[sl] + hv[sl]
            for f in range(_F):
                acc = acc + wb[f, sl] * hb[f, sl]
            rs[sl] = acc

    slots = ((wb0, hb0, wv0, hv0, rs0, isem0, osem0),
             (wb1, hb1, wv1, hv1, rs1, isem1, osem1))

    _fire(_in_descs(0, *slots[0][:4], slots[0][5]))
    _fire(_in_descs(1, *slots[1][:4], slots[1][5]))

    def step_slot(t, slot, fire_next, wait_prev_out):
        wb, hb, wv, hv, rs, isem, osem = slot
        _wait(_in_descs(t, wb, hb, wv, hv, isem))
        if wait_prev_out:
            _out_desc(t - 2, rs, osem).wait()
        _compute(wb, hb, wv, hv, rs)
        _out_desc(t, rs, osem).start()
        if fire_next:
            _fire(_in_descs(t + 2, wb, hb, wv, hv, isem))

    def outer(i, carry):
        t0 = i * 2
        for s in range(2):

            def _do_wait():
                _out_desc(t0 + s - 2, slots[s][4], slots[s][6]).wait()

            wbx, hbx, wvx, hvx, rsx, isemx, osemx = slots[s]
            _wait(_in_descs(t0 + s, wbx, hbx, wvx, hvx, isemx))
            pl.when(i > 0)(_do_wait)
            _compute(wbx, hbx, wvx, hvx, rsx)
            _out_desc(t0 + s, rsx, osemx).start()
            _fire(_in_descs(t0 + s + 2, wbx, hbx, wvx, hvx, isemx))
        return carry

    lax.fori_loop(0, _NSTEP // 2 - 1, outer, None)

    for s in range(2):
        t = _NSTEP - 2 + s
        step_slot(t, slots[s], fire_next=False, wait_prev_out=True)
    for s in range(2):
        _out_desc(_NSTEP - 2 + s, slots[s][4], slots[s][6]).wait()


_dot_scan_sc = pl.kernel(
    _scan_body,
    out_type=jax.ShapeDtypeStruct((_SCLEN,), jnp.float32),
    mesh=_mesh,
    compiler_params=pltpu.CompilerParams(use_tc_tiling_on_sc=True),
    scratch_types=[
        pltpu.VMEM((_F, _CH), jnp.float32),
        pltpu.VMEM((_F, _CH), jnp.float32),
        pltpu.VMEM((_CH,), jnp.float32),
        pltpu.VMEM((_CH,), jnp.float32),
        pltpu.VMEM((_CH,), jnp.float32),
        pltpu.VMEM((_F, _CH), jnp.float32),
        pltpu.VMEM((_F, _CH), jnp.float32),
        pltpu.VMEM((_CH,), jnp.float32),
        pltpu.VMEM((_CH,), jnp.float32),
        pltpu.VMEM((_CH,), jnp.float32),
        pltpu.SemaphoreType.DMA,
        pltpu.SemaphoreType.DMA,
        pltpu.SemaphoreType.DMA,
        pltpu.SemaphoreType.DMA,
    ],
)

_SCRATCH = [
    pltpu.VMEM((_NCHUNK, _ICH), jnp.int32),
    pltpu.VMEM((_NCHUNK, _ICH), jnp.int32),
    pltpu.VMEM((_NCHUNK, _ICH), jnp.int32),
    pltpu.VMEM((_NCHUNK, _ICH), jnp.int32),
    pltpu.VMEM((_BPW, _BROW), jnp.float32),
    pltpu.VMEM((_BPW, _BROW), jnp.float32),
    pltpu.VMEM((_BPW, _BROW), jnp.float32),
    pltpu.VMEM((_BPW,), jnp.float32),
    pltpu.SemaphoreType.DMA,
]


def _pick_body(nodes_hbm, da_hbm, db_hbm, dc_hbm, out_hbm,
               idx_v, diva_v, divb_v, divc_v,
               da_rows, db_rows, dc_rows, out_v, sem):
    wid = lax.axis_index("s") * _NC + lax.axis_index("c")

    pltpu.sync_copy(nodes_hbm.at[pl.ds(wid * _NCHUNK, _NCHUNK)], idx_v)

    for k in range(_NCHUNK):
        for c in range(_ICH // _L):
            sl = pl.ds(c * _L, _L)
            n = idx_v[k, sl]
            diva_v[k, sl] = lax.min(lax.shift_right_logical(n, 4),
                                    _NAROWS - 1)
            inb = jnp.where(n >= _SC0, 1, 0) * jnp.where(n < _SCEND, 1, 0)
            rb = lax.shift_right_logical(lax.max(n - _SC0, 0), 4)
            divb_v[k, sl] = inb * rb + (1 - inb) * jnp.bitwise_and(n, 16383)
            inc = jnp.where(n >= _SCEND, 1, 0)
            rc = lax.min(lax.shift_right_logical(lax.max(n - _SCEND, 0), 4),
                         _NCROWS - 1)
            divc_v[k, sl] = inc * rc + (1 - inc) * jnp.bitwise_and(n, 31)

    copies = []
    for k in range(_NCHUNK):
        rows = pl.ds(k * _ICH, _ICH)
        copies.append(pltpu.async_copy(da_hbm.at[diva_v.at[k]],
                                       da_rows.at[rows], sem))
        copies.append(pltpu.async_copy(db_hbm.at[divb_v.at[k]],
                                       db_rows.at[rows], sem))
        copies.append(pltpu.async_copy(dc_hbm.at[divc_v.at[k]],
                                       dc_rows.at[rows], sem))
    for c in copies:
        c.wait()

    lane = lax.iota(jnp.int32, _L)
    gdn = lax.GatherDimensionNumbers(
        offset_dims=(), collapsed_slice_dims=(0,), start_index_map=(0,))

    def _permute(v, perm2d):
        return lax.gather(v, perm2d, gdn, slice_sizes=(1,),
                          mode=lax.GatherScatterMode.PROMISE_IN_BOUNDS)

    bcast = [jnp.full((_L, 1), j, jnp.int32) for j in range(_L)]
    zero = jnp.zeros((_L,), jnp.float32)

    def group_body(g, carry):
        rbase = g * _L
        nid = idx_v[g // (_ICH // _L), pl.ds((g % (_ICH // _L)) * _L, _L)]
        col = jnp.bitwise_and(nid, _BROW - 1)
        in_b = (jnp.where(nid >= _SC0, 1, 0)
                * jnp.where(nid < _SCEND, 1, 0)).astype(jnp.float32)
        in_c = jnp.where(nid >= _SCEND, 1, 0).astype(jnp.float32)
        acc = zero
        for j in range(_L):
            rowa = da_rows[rbase + j, :]
            rowb = db_rows[rbase + j, :]
            rowc = dc_rows[rbase + j, :]
            bj = _permute(in_b, bcast[j])
            cj = _permute(in_c, bcast[j])
            srow = rowa + bj * (rowb - rowa) + cj * (rowc - rowa)
            colj = _permute(col, bcast[j])
            val = _permute(srow, colj[:, None])
            acc = jnp.where(lane == j, val, acc)
        out_v[pl.ds(rbase, _L)] = acc
        return carry

    lax.fori_loop(0, _BPW // _L, group_body, None)

    pltpu.sync_copy(out_v, out_hbm.at[pl.ds(wid * _BPW, _BPW)])


_pick_sc = pl.kernel(
    _pick_body,
    out_type=jax.ShapeDtypeStruct((_B,), jnp.float32),
    mesh=_mesh,
    compiler_params=pltpu.CompilerParams(use_tc_tiling_on_sc=False),
    scratch_types=_SCRATCH,
)


def kernel(nodes, W, H, w_bias, h_bias):
    wb1 = w_bias.reshape(-1)
    hb1 = h_bias.reshape(-1)
    wt = W.T
    ht = H.T
    dot_b = _dot_scan_sc(wt, ht, wb1, hb1)
    dot_a = _dot_scan_tc(wt, ht, wb1, hb1)
    dot_c = _dot_scan_tail(wt, ht, wb1, hb1)
    nodes2 = nodes.astype(jnp.int32).reshape(_NW * _NCHUNK, _ICH)
    return _pick_sc(nodes2,
                    dot_a.reshape(-1, _BROW),
                    dot_b.reshape(-1, _BROW),
                    dot_c.reshape(-1, _BROW))


# speedup vs baseline: 1.6821x; 1.6821x over previous
import jax
import jax.numpy as jnp
from jax import lax
from jax.experimental import pallas as pl
from jax.experimental.pallas import tpu as pltpu
from jax.experimental.pallas import tpu_sc as plsc

_B = 16384
_N = 1000000
_F = 32
_L = 16
_NC = 2
_NS = 16
_NW = _NC * _NS
_BPW = _B // _NW
_ICH = 128
_NCHUNK = _BPW // _ICH
_BROW = _L

_SCAN_BLK = 32768
_SCAN_GRID = (_N + _SCAN_BLK - 1) // _SCAN_BLK


def _dot_body(wt_ref, ht_ref, wb_ref, hb_ref, out_ref):
    out_ref[...] = (jnp.sum(wt_ref[...] * ht_ref[...], axis=0)
                    + wb_ref[...] + hb_ref[...])


_dot_scan = pl.pallas_call(
    _dot_body,
    out_shape=jax.ShapeDtypeStruct((_N,), jnp.float32),
    grid=(_SCAN_GRID,),
    in_specs=[
        pl.BlockSpec((_F, _SCAN_BLK), lambda i: (0, i)),
        pl.BlockSpec((_F, _SCAN_BLK), lambda i: (0, i)),
        pl.BlockSpec((_SCAN_BLK,), lambda i: (i,)),
        pl.BlockSpec((_SCAN_BLK,), lambda i: (i,)),
    ],
    out_specs=pl.BlockSpec((_SCAN_BLK,), lambda i: (i,)),
)

_mesh = plsc.VectorSubcoreMesh(core_axis_name="c", subcore_axis_name="s")

_SCRATCH = [
    pltpu.VMEM((_NCHUNK, _ICH), jnp.int32),
    pltpu.VMEM((_NCHUNK, _ICH), jnp.int32),
    pltpu.VMEM((_BPW, _BROW), jnp.float32),
    pltpu.VMEM((_BPW,), jnp.float32),
    pltpu.SemaphoreType.DMA,
]


def _pick_body(nodes_hbm, dt_hbm, out_hbm,
               idx_v, div_v, d_rows, out_v, sem):
    wid = lax.axis_index("s") * _NC + lax.axis_index("c")

    pltpu.sync_copy(nodes_hbm.at[pl.ds(wid * _NCHUNK, _NCHUNK)], idx_v)

    for k in range(_NCHUNK):
        for c in range(_ICH // _L):
            sl = pl.ds(c * _L, _L)
            div_v[k, sl] = lax.shift_right_logical(idx_v[k, sl], 4)

    copies = []
    for k in range(_NCHUNK):
        rows = pl.ds(k * _ICH, _ICH)
        copies.append(pltpu.async_copy(dt_hbm.at[div_v.at[k]],
                                       d_rows.at[rows], sem))
    for c in copies:
        c.wait()

    lane = lax.iota(jnp.int32, _L)
    gdn = lax.GatherDimensionNumbers(
        offset_dims=(), collapsed_slice_dims=(0,), start_index_map=(0,))

    def _permute(v, perm2d):
        return lax.gather(v, perm2d, gdn, slice_sizes=(1,),
                          mode=lax.GatherScatterMode.PROMISE_IN_BOUNDS)

    bcast = [jnp.full((_L, 1), j, jnp.int32) for j in range(_L)]
    zero = jnp.zeros((_L,), jnp.float32)

    def group_body(g, carry):
        rbase = g * _L
        nid = idx_v[g // (_ICH // _L), pl.ds((g % (_ICH // _L)) * _L, _L)]
        col = jnp.bitwise_and(nid, _BROW - 1)
        acc = zero
        for j in range(_L):
            srow = d_rows[rbase + j, :]
            colj = _permute(col, bcast[j])
            val = _permute(srow, colj[:, None])
            acc = jnp.where(lane == j, val, acc)
        out_v[pl.ds(rbase, _L)] = acc
        return carry

    lax.fori_loop(0, _BPW // _L, group_body, None)

    pltpu.sync_copy(out_v, out_hbm.at[pl.ds(wid * _BPW, _BPW)])


_pick_sc = pl.kernel(
    _pick_body,
    out_type=jax.ShapeDtypeStruct((_B,), jnp.float32),
    mesh=_mesh,
    compiler_params=pltpu.CompilerParams(use_tc_tiling_on_sc=False),
    scratch_types=_SCRATCH,
)


def kernel(nodes, W, H, w_bias, h_bias):
    dotall = _dot_scan(W.T, H.T, w_bias.reshape(-1), h_bias.reshape(-1))
    nodes2 = nodes.astype(jnp.int32).reshape(_NW * _NCHUNK, _ICH)
    return _pick_sc(nodes2, dotall.reshape(-1, _BROW))
